# baseline (speedup 1.0000x reference)
import functools

import jax
import jax.numpy as jnp
from jax import lax
from jax.experimental import pallas as pl
from jax.experimental.pallas import tpu as pltpu
from jax.experimental.pallas import tpu_sc as plsc

_NUM_FEATURES = 26
_BATCH = 16384
_N = _NUM_FEATURES * _BATCH
_ZCH = 250000
_ROWS = _ZCH // 16
_CPAD = 16384
_IMAX = 2147483647

_NC = 2
_NS = 16

_CHUNK = 1024
_SUB = 128
_NSUB = _CHUNK // _SUB
_ILP = 4

_SEG01 = 7 * _BATCH
_PT01 = _SEG01 // 8
_PT23 = (6 * _BATCH) // 8


def _sc_body(values_hbm, tables_hbm, coarse_hbm, out_hbm,
             coarse_v, vals_v, bidx_v, rows_v, out_v, sem):
  cid = lax.axis_index("c")
  sid = lax.axis_index("s")
  wid = sid * _NC + cid
  tid = wid >> 3
  k = wid & 7

  is01 = tid < 2
  per_tile = jnp.where(is01, _PT01, _PT23)
  seg_start = jnp.where(tid == 0, 0,
                        jnp.where(tid == 1, _SEG01,
                                  jnp.where(tid == 2, 2 * _SEG01,
                                            2 * _SEG01 + 6 * _BATCH)))
  base = seg_start + k * per_tile
  nchunks = jnp.where(is01, _PT01 // _CHUNK, _PT23 // _CHUNK)
  row_base = tid * _ROWS

  pltpu.sync_copy(coarse_hbm.at[pl.ds(tid * _CPAD, _CPAD)], coarse_v)

  def chunk_body(ci, carry):
    off = base + ci * _CHUNK
    pltpu.sync_copy(values_hbm.at[pl.ds(off, _CHUNK)], vals_v)

    def grp1(g, carry):
      voff = g * (16 * _ILP)
      v = [vals_v[pl.ds(voff + 16 * u, 16)] for u in range(_ILP)]
      p1 = [jnp.full((16,), -1, jnp.int32) for _ in range(_ILP)]
      for s in range(14):
        half = _CPAD >> (s + 1)
        probe = [p1[u] + half for u in range(_ILP)]
        keys = [plsc.load_gather(coarse_v, [probe[u]]) for u in range(_ILP)]
        p1 = [jnp.where(keys[u] < v[u], probe[u], p1[u]) for u in range(_ILP)]
      for u in range(_ILP):
        b = jnp.minimum(p1[u] + 1, _ROWS - 1)
        bidx_v[pl.ds(voff + 16 * u, 16)] = b + row_base
      return carry

    lax.fori_loop(0, _CHUNK // (16 * _ILP), grp1, 0)

    copies = [
        pltpu.async_copy(tables_hbm.at[bidx_v.at[pl.ds(j * _SUB, _SUB)]],
                         rows_v.at[pl.ds(j * _SUB, _SUB)], sem)
        for j in range(_NSUB)
    ]
    for c in copies:
      c.wait()

    def grp3(g, carry):
      voff = g * (16 * _ILP)
      iota = lax.iota(jnp.int32, 16)
      for u in range(_ILP):
        v = vals_v[pl.ds(voff + 16 * u, 16)]
        b = bidx_v[pl.ds(voff + 16 * u, 16)] - row_base
        rv = voff + 16 * u + iota
        q1 = jnp.full((16,), -1, jnp.int32)
        for s in range(4):
          half = 8 >> s
          probe = q1 + half
          keys = plsc.load_gather(rows_v, [rv, probe])
          q1 = jnp.where(keys < v, probe, q1)
        pos2 = q1 + 1
        row_at = plsc.load_gather(rows_v, [rv, jnp.minimum(pos2, 15)])
        hit = (pos2 < 16) & (row_at == v)
        miss = lax.rem(v, jnp.full((16,), _ZCH, jnp.int32))
        out_v[pl.ds(voff + 16 * u, 16)] = jnp.where(hit, b * 16 + pos2, miss)
      return carry

    lax.fori_loop(0, _CHUNK // (16 * _ILP), grp3, 0)

    pltpu.sync_copy(out_v, out_hbm.at[pl.ds(off, _CHUNK)])
    return carry

  lax.fori_loop(0, nchunks, chunk_body, 0)


_sc_remap = functools.partial(
    pl.kernel,
    out_type=jax.ShapeDtypeStruct((_N,), jnp.int32),
    mesh=plsc.VectorSubcoreMesh(core_axis_name="c", subcore_axis_name="s"),
    scratch_types=[
        pltpu.VMEM((_CPAD,), jnp.int32),
        pltpu.VMEM((_CHUNK,), jnp.int32),
        pltpu.VMEM((_CHUNK,), jnp.int32),
        pltpu.VMEM((_CHUNK, 16), jnp.int32),
        pltpu.VMEM((_CHUNK,), jnp.int32),
        pltpu.SemaphoreType.DMA,
    ],
    compiler_params=pltpu.CompilerParams(needs_layout_passes=False,
                                         use_tc_tiling_on_sc=False),
)(_sc_body)


def kernel(values, lengths, weights, mch_sorted_ids_0, mch_sorted_ids_1,
           mch_sorted_ids_2, mch_sorted_ids_3):
  tables = jnp.stack([mch_sorted_ids_0, mch_sorted_ids_1,
                      mch_sorted_ids_2, mch_sorted_ids_3])
  tables2d = tables.reshape(4 * _ROWS, 16)
  coarse = jnp.max(tables.reshape(4, _ROWS, 16), axis=2)
  coarse = jnp.pad(coarse, ((0, 0), (0, _CPAD - _ROWS)),
                   constant_values=_IMAX).reshape(-1)
  out_values = _sc_remap(values, tables2d, coarse)
  return out_values, lengths, weights


# speedup vs baseline: 1.0424x; 1.0424x over previous
import functools

import jax
import jax.numpy as jnp
from jax import lax
from jax.experimental import pallas as pl
from jax.experimental.pallas import tpu as pltpu
from jax.experimental.pallas import tpu_sc as plsc

_NUM_FEATURES = 26
_BATCH = 16384
_N = _NUM_FEATURES * _BATCH
_ZCH = 250000
_ROWS = _ZCH // 16
_CPAD = 16384
_IMAX = 2147483647

_NC = 2
_NS = 16

_CHUNK = 1024
_SUB = 128
_NSUB = _CHUNK // _SUB
_ILP = 4

_SEG01 = 7 * _BATCH
_PT01 = _SEG01 // 8
_PT23 = (6 * _BATCH) // 8


def _sc_body(values_hbm, tables_hbm, coarse_hbm, out_hbm,
             coarse_v, vals_v, bidx_v, rows_v, out_v, sem):
  tables2d = tables_hbm
  cid = lax.axis_index("c")
  sid = lax.axis_index("s")
  wid = sid * _NC + cid
  tid = wid >> 3
  k = wid & 7

  is01 = tid < 2
  per_tile = jnp.where(is01, _PT01, _PT23)
  seg_start = jnp.where(tid == 0, 0,
                        jnp.where(tid == 1, _SEG01,
                                  jnp.where(tid == 2, 2 * _SEG01,
                                            2 * _SEG01 + 6 * _BATCH)))
  base = seg_start + k * per_tile
  nchunks = jnp.where(is01, _PT01 // _CHUNK, _PT23 // _CHUNK)
  row_base = tid * _CPAD

  pltpu.sync_copy(coarse_hbm.at[pl.ds(tid * _CPAD, _CPAD)], coarse_v)

  def chunk_body(ci, carry):
    off = base + ci * _CHUNK
    pltpu.sync_copy(values_hbm.at[pl.ds(off, _CHUNK)], vals_v)

    def grp1(g, carry):
      voff = g * (16 * _ILP)
      v = [vals_v[pl.ds(voff + 16 * u, 16)] for u in range(_ILP)]
      p1 = [jnp.full((16,), -1, jnp.int32) for _ in range(_ILP)]
      for s in range(14):
        half = _CPAD >> (s + 1)
        probe = [p1[u] + half for u in range(_ILP)]
        keys = [plsc.load_gather(coarse_v, [probe[u]]) for u in range(_ILP)]
        p1 = [jnp.where(keys[u] < v[u], probe[u], p1[u]) for u in range(_ILP)]
      for u in range(_ILP):
        b = jnp.minimum(p1[u] + 1, _ROWS - 1)
        bidx_v[pl.ds(voff + 16 * u, 16)] = b + row_base
      return carry

    lax.fori_loop(0, _CHUNK // (16 * _ILP), grp1, 0)

    copies = [
        pltpu.async_copy(tables2d.at[bidx_v.at[pl.ds(j * _SUB, _SUB)]],
                         rows_v.at[pl.ds(j * _SUB, _SUB)], sem)
        for j in range(_NSUB)
    ]
    for c in copies:
      c.wait()

    def grp3(g, carry):
      voff = g * (16 * _ILP)
      iota = lax.iota(jnp.int32, 16)
      for u in range(_ILP):
        v = vals_v[pl.ds(voff + 16 * u, 16)]
        b = bidx_v[pl.ds(voff + 16 * u, 16)] - row_base
        rv = voff + 16 * u + iota
        q1 = jnp.full((16,), -1, jnp.int32)
        for s in range(4):
          half = 8 >> s
          probe = q1 + half
          keys = plsc.load_gather(rows_v, [rv, probe])
          q1 = jnp.where(keys < v, probe, q1)
        pos2 = q1 + 1
        row_at = plsc.load_gather(rows_v, [rv, jnp.minimum(pos2, 15)])
        hit = (pos2 < 16) & (row_at == v)
        miss = lax.rem(v, jnp.full((16,), _ZCH, jnp.int32))
        out_v[pl.ds(voff + 16 * u, 16)] = jnp.where(hit, b * 16 + pos2, miss)
      return carry

    lax.fori_loop(0, _CHUNK // (16 * _ILP), grp3, 0)

    pltpu.sync_copy(out_v, out_hbm.at[pl.ds(off, _CHUNK)])
    return carry

  lax.fori_loop(0, nchunks, chunk_body, 0)


_sc_remap = functools.partial(
    pl.kernel,
    out_type=jax.ShapeDtypeStruct((_N,), jnp.int32),
    mesh=plsc.VectorSubcoreMesh(core_axis_name="c", subcore_axis_name="s"),
    scratch_types=[
        pltpu.VMEM((_CPAD,), jnp.int32),
        pltpu.VMEM((_CHUNK,), jnp.int32),
        pltpu.VMEM((_CHUNK,), jnp.int32),
        pltpu.VMEM((_CHUNK, 16), jnp.int32),
        pltpu.VMEM((_CHUNK,), jnp.int32),
        pltpu.SemaphoreType.DMA,
    ],
    compiler_params=pltpu.CompilerParams(needs_layout_passes=False,
                                         use_tc_tiling_on_sc=False),
)(_sc_body)


def kernel(values, lengths, weights, mch_sorted_ids_0, mch_sorted_ids_1,
           mch_sorted_ids_2, mch_sorted_ids_3):
  pad = jnp.full(((_CPAD - _ROWS) * 16,), _IMAX, jnp.int32)
  tables_flat = jnp.concatenate([mch_sorted_ids_0, pad, mch_sorted_ids_1, pad,
                                 mch_sorted_ids_2, pad, mch_sorted_ids_3, pad])
  tables2d = tables_flat.reshape(4 * _CPAD, 16)
  coarse = jnp.max(tables2d, axis=1)
  out_values = _sc_remap(values, tables2d, coarse)
  return out_values, lengths, weights


# speedup vs baseline: 1.3072x; 1.2539x over previous
import functools

import jax
import jax.numpy as jnp
from jax import lax
from jax.experimental import pallas as pl
from jax.experimental.pallas import tpu as pltpu
from jax.experimental.pallas import tpu_sc as plsc

_NUM_FEATURES = 26
_BATCH = 16384
_N = _NUM_FEATURES * _BATCH
_ZCH = 250000
_ROWS = _ZCH // 16
_CPAD = 16384
_IMAX = 2147483647

_NC = 2
_NS = 16

_CHUNK = 1024
_SUB = 128
_NSUB = _CHUNK // _SUB
_ILP = 4

_SPAN = 1952
_SPAN7 = _ROWS - 7 * _SPAN
_CFILL = 15632

_SEG01 = 7 * _BATCH
_PT01 = _SEG01 // 8
_PT23 = (6 * _BATCH) // 8


def _sc_body(values_hbm, m0, m1, m2, m3, out_hbm,
             coarse_v, stage_v, vals_v, bidx_v, rows_v, out_v, spmem, sem):
  tabs = [m0, m1, m2, m3]
  cid = lax.axis_index("c")
  sid = lax.axis_index("s")
  wid = cid * _NS + sid
  tid = wid >> 3
  k = wid & 7
  g = tid & 1

  is01 = tid < 2
  per_tile = jnp.where(is01, _PT01, _PT23)
  seg_start = jnp.where(tid == 0, 0,
                        jnp.where(tid == 1, _SEG01,
                                  jnp.where(tid == 2, 2 * _SEG01,
                                            2 * _SEG01 + 6 * _BATCH)))
  base = seg_start + k * per_tile
  nchunks = jnp.where(is01, _PT01 // _CHUNK, _PT23 // _CHUNK)

  iota = lax.iota(jnp.int32, 16)
  c15 = jnp.full((16,), 15, jnp.int32)
  imax_vec = jnp.full((16,), _IMAX, jnp.int32)

  def extract(m):
    def ex():
      row0 = k * _SPAN

      @pl.when(k < 7)
      def _():
        pltpu.sync_copy(m.at[pl.ds(row0, _SPAN), :],
                        stage_v.at[pl.ds(0, _SPAN), :])

      @pl.when(k == 7)
      def _():
        pltpu.sync_copy(m.at[pl.ds(row0, _SPAN7), :], stage_v)

      nu = jnp.where(k == 7, _SPAN7 // 16 + 1, _SPAN // 16)

      def gx(u, carry):
        keys = plsc.load_gather(stage_v, [u * 16 + iota, c15])
        coarse_v[pl.ds(row0 + u * 16, 16)] = keys
        return carry

      lax.fori_loop(0, nu, gx, 0)

      @pl.when(k == 7)
      def _():
        last = coarse_v[pl.ds(_CFILL - 16, 16)]
        coarse_v[pl.ds(_CFILL - 16, 16)] = jnp.where(
            iota < _ROWS - (_CFILL - 16), last, imax_vec)
        pltpu.sync_copy(coarse_v.at[pl.ds(row0, _CFILL - 7 * _SPAN)],
                        spmem.at[g, pl.ds(row0, _CFILL - 7 * _SPAN)])

      @pl.when(k < 7)
      def _():
        pltpu.sync_copy(coarse_v.at[pl.ds(row0, _SPAN)],
                        spmem.at[g, pl.ds(row0, _SPAN)])
    return ex

  for t in range(4):
    pl.when(tid == t)(extract(tabs[t]))

  plsc.subcore_barrier()

  pltpu.sync_copy(spmem.at[g, pl.ds(0, _CFILL)], coarse_v.at[pl.ds(0, _CFILL)])

  def fill(i, carry):
    coarse_v[pl.ds(_CFILL + i * 16, 16)] = imax_vec
    return carry

  lax.fori_loop(0, (_CPAD - _CFILL) // 16, fill, 0)

  def main_loop(m):
    def run():
      def chunk_body(ci, carry):
        off = base + ci * _CHUNK
        pltpu.sync_copy(values_hbm.at[pl.ds(off, _CHUNK)], vals_v)

        def grp1(gi, carry):
          voff = gi * (16 * _ILP)
          v = [vals_v[pl.ds(voff + 16 * u, 16)] for u in range(_ILP)]
          p1 = [jnp.full((16,), -1, jnp.int32) for _ in range(_ILP)]
          for s in range(14):
            half = _CPAD >> (s + 1)
            probe = [p1[u] + half for u in range(_ILP)]
            keys = [plsc.load_gather(coarse_v, [probe[u]])
                    for u in range(_ILP)]
            p1 = [jnp.where(keys[u] < v[u], probe[u], p1[u])
                  for u in range(_ILP)]
          for u in range(_ILP):
            bidx_v[pl.ds(voff + 16 * u, 16)] = jnp.minimum(p1[u] + 1,
                                                           _ROWS - 1)
          return carry

        lax.fori_loop(0, _CHUNK // (16 * _ILP), grp1, 0)

        copies = [
            pltpu.async_copy(m.at[bidx_v.at[pl.ds(j * _SUB, _SUB)]],
                             rows_v.at[pl.ds(j * _SUB, _SUB)], sem)
            for j in range(_NSUB)
        ]
        for c in copies:
          c.wait()

        def grp3(gi, carry):
          voff = gi * (16 * _ILP)
          for u in range(_ILP):
            v = vals_v[pl.ds(voff + 16 * u, 16)]
            b = bidx_v[pl.ds(voff + 16 * u, 16)]
            rv = voff + 16 * u + iota
            q1 = jnp.full((16,), -1, jnp.int32)
            for s in range(4):
              half = 8 >> s
              probe = q1 + half
              keys = plsc.load_gather(rows_v, [rv, probe])
              q1 = jnp.where(keys < v, probe, q1)
            pos2 = q1 + 1
            row_at = plsc.load_gather(rows_v, [rv, jnp.minimum(pos2, 15)])
            hit = (pos2 < 16) & (row_at == v)
            miss = lax.rem(v, jnp.full((16,), _ZCH, jnp.int32))
            out_v[pl.ds(voff + 16 * u, 16)] = jnp.where(hit, b * 16 + pos2,
                                                        miss)
          return carry

        lax.fori_loop(0, _CHUNK // (16 * _ILP), grp3, 0)

        pltpu.sync_copy(out_v, out_hbm.at[pl.ds(off, _CHUNK)])
        return carry

      lax.fori_loop(0, nchunks, chunk_body, 0)
    return run

  for t in range(4):
    pl.when(tid == t)(main_loop(tabs[t]))


_sc_remap = functools.partial(
    pl.kernel,
    out_type=jax.ShapeDtypeStruct((_N,), jnp.int32),
    mesh=plsc.VectorSubcoreMesh(core_axis_name="c", subcore_axis_name="s"),
    scratch_types=[
        pltpu.VMEM((_CPAD,), jnp.int32),
        pltpu.VMEM((_SPAN7, 16), jnp.int32),
        pltpu.VMEM((_CHUNK,), jnp.int32),
        pltpu.VMEM((_CHUNK,), jnp.int32),
        pltpu.VMEM((_CHUNK, 16), jnp.int32),
        pltpu.VMEM((_CHUNK,), jnp.int32),
        pltpu.VMEM_SHARED((2, _CPAD), jnp.int32),
        pltpu.SemaphoreType.DMA,
    ],
    compiler_params=pltpu.CompilerParams(needs_layout_passes=False,
                                         use_tc_tiling_on_sc=False),
)(_sc_body)


def kernel(values, lengths, weights, mch_sorted_ids_0, mch_sorted_ids_1,
           mch_sorted_ids_2, mch_sorted_ids_3):
  out_values = _sc_remap(values,
                         mch_sorted_ids_0.reshape(_ROWS, 16),
                         mch_sorted_ids_1.reshape(_ROWS, 16),
                         mch_sorted_ids_2.reshape(_ROWS, 16),
                         mch_sorted_ids_3.reshape(_ROWS, 16))
  return out_values, lengths, weights


# speedup vs baseline: 1.4658x; 1.1213x over previous
import functools

import jax
import jax.numpy as jnp
from jax import lax
from jax.experimental import pallas as pl
from jax.experimental.pallas import tpu as pltpu
from jax.experimental.pallas import tpu_sc as plsc

_NUM_FEATURES = 26
_BATCH = 16384
_N = _NUM_FEATURES * _BATCH
_ZCH = 250000
_ROWS = _ZCH // 16
_CPAD = 16384
_IMAX = 2147483647

_NC = 2
_NS = 16

_CHUNK = 1024
_SUB = 128
_NSUB = _CHUNK // _SUB
_ILP = 4

_SPAN = 1952
_SPAN7 = _ROWS - 7 * _SPAN
_CFILL = 15632

_SEG01 = 7 * _BATCH
_PT01 = _SEG01 // 8
_PT23 = (6 * _BATCH) // 8


def _sc_body(values_hbm, m0, m1, m2, m3, out_hbm,
             coarse_v, stage_v, vals_v, bidx_v, rows_v, out_v, spmem,
             sem, sem_vals, sem_out):
  tabs = [m0, m1, m2, m3]
  cid = lax.axis_index("c")
  sid = lax.axis_index("s")
  wid = cid * _NS + sid
  tid = wid >> 3
  k = wid & 7
  g = tid & 1

  is01 = tid < 2
  per_tile = jnp.where(is01, _PT01, _PT23)
  seg_start = jnp.where(tid == 0, 0,
                        jnp.where(tid == 1, _SEG01,
                                  jnp.where(tid == 2, 2 * _SEG01,
                                            2 * _SEG01 + 6 * _BATCH)))
  base = seg_start + k * per_tile
  nchunks = jnp.where(is01, _PT01 // _CHUNK, _PT23 // _CHUNK)

  iota = lax.iota(jnp.int32, 16)
  c15 = jnp.full((16,), 15, jnp.int32)
  imax_vec = jnp.full((16,), _IMAX, jnp.int32)

  def extract(m):
    def ex():
      row0 = k * _SPAN

      @pl.when(k < 7)
      def _():
        pltpu.sync_copy(m.at[pl.ds(row0, _SPAN), :],
                        stage_v.at[pl.ds(0, _SPAN), :])

      @pl.when(k == 7)
      def _():
        pltpu.sync_copy(m.at[pl.ds(row0, _SPAN7), :], stage_v)

      nu = jnp.where(k == 7, _SPAN7 // 16 + 1, _SPAN // 16)

      def gx(u, carry):
        keys = plsc.load_gather(stage_v, [u * 16 + iota, c15])
        coarse_v[pl.ds(row0 + u * 16, 16)] = keys
        return carry

      lax.fori_loop(0, nu, gx, 0)

      @pl.when(k == 7)
      def _():
        last = coarse_v[pl.ds(_CFILL - 16, 16)]
        coarse_v[pl.ds(_CFILL - 16, 16)] = jnp.where(
            iota < _ROWS - (_CFILL - 16), last, imax_vec)
        pltpu.sync_copy(coarse_v.at[pl.ds(row0, _CFILL - 7 * _SPAN)],
                        spmem.at[g, pl.ds(row0, _CFILL - 7 * _SPAN)])

      @pl.when(k < 7)
      def _():
        pltpu.sync_copy(coarse_v.at[pl.ds(row0, _SPAN)],
                        spmem.at[g, pl.ds(row0, _SPAN)])
    return ex

  for t in range(4):
    pl.when(tid == t)(extract(tabs[t]))

  plsc.subcore_barrier()

  pltpu.sync_copy(spmem.at[g, pl.ds(0, _CFILL)], coarse_v.at[pl.ds(0, _CFILL)])

  def fill(i, carry):
    coarse_v[pl.ds(_CFILL + i * 16, 16)] = imax_vec
    return carry

  lax.fori_loop(0, (_CPAD - _CFILL) // 16, fill, 0)

  def main_loop(m):
    def phase1(b):
      def grp1(gi, carry):
        voff = gi * (16 * _ILP)
        v = [vals_v[b, pl.ds(voff + 16 * u, 16)] for u in range(_ILP)]
        p1 = [jnp.full((16,), -1, jnp.int32) for _ in range(_ILP)]
        for s in range(14):
          half = _CPAD >> (s + 1)
          probe = [p1[u] + half for u in range(_ILP)]
          keys = [plsc.load_gather(coarse_v, [probe[u]])
                  for u in range(_ILP)]
          p1 = [jnp.where(keys[u] < v[u], probe[u], p1[u])
                for u in range(_ILP)]
        for u in range(_ILP):
          bidx_v[b, pl.ds(voff + 16 * u, 16)] = jnp.minimum(p1[u] + 1,
                                                            _ROWS - 1)
        return carry

      lax.fori_loop(0, _CHUNK // (16 * _ILP), grp1, 0)

    def fire_rows(b):
      for j in range(_NSUB):
        pltpu.async_copy(m.at[bidx_v.at[b, pl.ds(j * _SUB, _SUB)]],
                         rows_v.at[b, pl.ds(j * _SUB, _SUB)], sem)

    def drain_rows(b):
      for j in range(_NSUB):
        pltpu.make_async_copy(m.at[bidx_v.at[b, pl.ds(j * _SUB, _SUB)]],
                              rows_v.at[b, pl.ds(j * _SUB, _SUB)],
                              sem).wait()

    def phase3(ci, b):
      bb = jnp.full((16,), b, jnp.int32)

      def grp3(gi, carry):
        voff = gi * (16 * _ILP)
        for u in range(_ILP):
          v = vals_v[b, pl.ds(voff + 16 * u, 16)]
          bk = bidx_v[b, pl.ds(voff + 16 * u, 16)]
          rv = voff + 16 * u + iota
          q1 = jnp.full((16,), -1, jnp.int32)
          for s in range(4):
            half = 8 >> s
            probe = q1 + half
            keys = plsc.load_gather(rows_v, [bb, rv, probe])
            q1 = jnp.where(keys < v, probe, q1)
          pos2 = q1 + 1
          row_at = plsc.load_gather(rows_v, [bb, rv, jnp.minimum(pos2, 15)])
          hit = (pos2 < 16) & (row_at == v)
          miss = lax.rem(v, jnp.full((16,), _ZCH, jnp.int32))
          out_v[b, pl.ds(voff + 16 * u, 16)] = jnp.where(hit, bk * 16 + pos2,
                                                         miss)
        return carry

      lax.fori_loop(0, _CHUNK // (16 * _ILP), grp3, 0)

    def fire_vals(ci, b):
      pltpu.async_copy(values_hbm.at[pl.ds(base + ci * _CHUNK, _CHUNK)],
                       vals_v.at[b], sem_vals)

    def wait_vals(ci, b):
      pltpu.make_async_copy(values_hbm.at[pl.ds(base + ci * _CHUNK, _CHUNK)],
                            vals_v.at[b], sem_vals).wait()

    def fire_out(ci, b):
      pltpu.async_copy(out_v.at[b],
                       out_hbm.at[pl.ds(base + ci * _CHUNK, _CHUNK)], sem_out)

    def drain_out(ci, b):
      pltpu.make_async_copy(out_v.at[b],
                            out_hbm.at[pl.ds(base + ci * _CHUNK, _CHUNK)],
                            sem_out).wait()

    def run():
      pltpu.sync_copy(values_hbm.at[pl.ds(base, _CHUNK)], vals_v.at[0])
      fire_vals(1, 1)
      phase1(0)
      fire_rows(0)

      def body(ci, carry):
        b0 = ci & 1
        b1 = 1 - b0
        wait_vals(ci + 1, b1)
        phase1(b1)
        fire_rows(b1)
        drain_rows(b0)

        @pl.when(ci >= 2)
        def _():
          drain_out(ci, b0)

        phase3(ci, b0)
        fire_out(ci, b0)

        @pl.when(ci + 2 <= nchunks - 1)
        def _():
          fire_vals(ci + 2, b0)

        return carry

      lax.fori_loop(0, nchunks - 1, body, 0)

      bl = (nchunks - 1) & 1
      drain_rows(bl)

      @pl.when(nchunks >= 3)
      def _():
        drain_out(nchunks - 1, bl)

      phase3(nchunks - 1, bl)
      fire_out(nchunks - 1, bl)
      drain_out(nchunks - 2, 1 - bl)
      drain_out(nchunks - 1, bl)
    return run

  for t in range(4):
    pl.when(tid == t)(main_loop(tabs[t]))


_sc_remap = functools.partial(
    pl.kernel,
    out_type=jax.ShapeDtypeStruct((_N,), jnp.int32),
    mesh=plsc.VectorSubcoreMesh(core_axis_name="c", subcore_axis_name="s"),
    scratch_types=[
        pltpu.VMEM((_CPAD,), jnp.int32),
        pltpu.VMEM((_SPAN7, 16), jnp.int32),
        pltpu.VMEM((2, _CHUNK), jnp.int32),
        pltpu.VMEM((2, _CHUNK), jnp.int32),
        pltpu.VMEM((2, _CHUNK, 16), jnp.int32),
        pltpu.VMEM((2, _CHUNK), jnp.int32),
        pltpu.VMEM_SHARED((2, _CPAD), jnp.int32),
        pltpu.SemaphoreType.DMA,
        pltpu.SemaphoreType.DMA,
        pltpu.SemaphoreType.DMA,
    ],
    compiler_params=pltpu.CompilerParams(needs_layout_passes=False,
                                         use_tc_tiling_on_sc=False),
)(_sc_body)


def kernel(values, lengths, weights, mch_sorted_ids_0, mch_sorted_ids_1,
           mch_sorted_ids_2, mch_sorted_ids_3):
  out_values = _sc_remap(values,
                         mch_sorted_ids_0.reshape(_ROWS, 16),
                         mch_sorted_ids_1.reshape(_ROWS, 16),
                         mch_sorted_ids_2.reshape(_ROWS, 16),
                         mch_sorted_ids_3.reshape(_ROWS, 16))
  return out_values, lengths, weights


# speedup vs baseline: 1.5153x; 1.0338x over previous
import functools

import jax
import jax.numpy as jnp
from jax import lax
from jax.experimental import pallas as pl
from jax.experimental.pallas import tpu as pltpu
from jax.experimental.pallas import tpu_sc as plsc

_NUM_FEATURES = 26
_BATCH = 16384
_N = _NUM_FEATURES * _BATCH
_ZCH = 250000
_ROWS = _ZCH // 16
_CPAD = 16384
_IMAX = 2147483647

_NC = 2
_NS = 16

_CHUNK = 1024
_SUB = 128
_NSUB = _CHUNK // _SUB
_ILP = 8

_SPAN = 1952
_SPAN7 = _ROWS - 7 * _SPAN
_CFILL = 15632

_SEG01 = 7 * _BATCH
_PT01 = _SEG01 // 8
_PT23 = (6 * _BATCH) // 8


def _sc_body(values_hbm, m0, m1, m2, m3, out_hbm,
             coarse_v, stage_v, vals_v, bidx_v, rows_v, out_v, spmem,
             sem, sem_vals, sem_out):
  tabs = [m0, m1, m2, m3]
  cid = lax.axis_index("c")
  sid = lax.axis_index("s")
  wid = cid * _NS + sid
  tid = wid >> 3
  k = wid & 7
  g = tid & 1

  is01 = tid < 2
  per_tile = jnp.where(is01, _PT01, _PT23)
  seg_start = jnp.where(tid == 0, 0,
                        jnp.where(tid == 1, _SEG01,
                                  jnp.where(tid == 2, 2 * _SEG01,
                                            2 * _SEG01 + 6 * _BATCH)))
  base = seg_start + k * per_tile
  nchunks = jnp.where(is01, _PT01 // _CHUNK, _PT23 // _CHUNK)

  iota = lax.iota(jnp.int32, 16)
  c15 = jnp.full((16,), 15, jnp.int32)
  imax_vec = jnp.full((16,), _IMAX, jnp.int32)

  def extract(m):
    def ex():
      row0 = k * _SPAN

      @pl.when(k < 7)
      def _():
        pltpu.sync_copy(m.at[pl.ds(row0, _SPAN), :],
                        stage_v.at[pl.ds(0, _SPAN), :])

      @pl.when(k == 7)
      def _():
        pltpu.sync_copy(m.at[pl.ds(row0, _SPAN7), :], stage_v)

      nu = jnp.where(k == 7, _SPAN7 // 16 + 1, _SPAN // 16)

      def gx(u, carry):
        keys = plsc.load_gather(stage_v, [u * 16 + iota, c15])
        coarse_v[pl.ds(row0 + u * 16, 16)] = keys
        return carry

      lax.fori_loop(0, nu, gx, 0)

      @pl.when(k == 7)
      def _():
        last = coarse_v[pl.ds(_CFILL - 16, 16)]
        coarse_v[pl.ds(_CFILL - 16, 16)] = jnp.where(
            iota < _ROWS - (_CFILL - 16), last, imax_vec)
        pltpu.sync_copy(coarse_v.at[pl.ds(row0, _CFILL - 7 * _SPAN)],
                        spmem.at[g, pl.ds(row0, _CFILL - 7 * _SPAN)])

      @pl.when(k < 7)
      def _():
        pltpu.sync_copy(coarse_v.at[pl.ds(row0, _SPAN)],
                        spmem.at[g, pl.ds(row0, _SPAN)])
    return ex

  for t in range(4):
    pl.when(tid == t)(extract(tabs[t]))

  plsc.subcore_barrier()

  pltpu.sync_copy(spmem.at[g, pl.ds(0, _CFILL)], coarse_v.at[pl.ds(0, _CFILL)])

  def fill(i, carry):
    coarse_v[pl.ds(_CFILL + i * 16, 16)] = imax_vec
    return carry

  lax.fori_loop(0, (_CPAD - _CFILL) // 16, fill, 0)

  def main_loop(m):
    def phase1(b):
      def grp1(gi, carry):
        voff = gi * (16 * _ILP)
        v = [vals_v[b, pl.ds(voff + 16 * u, 16)] for u in range(_ILP)]
        p1 = [jnp.full((16,), -1, jnp.int32) for _ in range(_ILP)]
        for s in range(14):
          half = _CPAD >> (s + 1)
          probe = [p1[u] + half for u in range(_ILP)]
          keys = [plsc.load_gather(coarse_v, [probe[u]])
                  for u in range(_ILP)]
          p1 = [jnp.where(keys[u] < v[u], probe[u], p1[u])
                for u in range(_ILP)]
        for u in range(_ILP):
          bidx_v[b, pl.ds(voff + 16 * u, 16)] = jnp.minimum(p1[u] + 1,
                                                            _ROWS - 1)
        return carry

      lax.fori_loop(0, _CHUNK // (16 * _ILP), grp1, 0)

    def fire_rows(b):
      for j in range(_NSUB):
        pltpu.async_copy(m.at[bidx_v.at[b, pl.ds(j * _SUB, _SUB)]],
                         rows_v.at[b, pl.ds(j * _SUB, _SUB)], sem)

    def drain_rows(b):
      for j in range(_NSUB):
        pltpu.make_async_copy(m.at[bidx_v.at[b, pl.ds(j * _SUB, _SUB)]],
                              rows_v.at[b, pl.ds(j * _SUB, _SUB)],
                              sem).wait()

    def phase3(ci, b):
      bb = jnp.full((16,), b, jnp.int32)

      def grp3(gi, carry):
        voff = gi * (16 * _ILP)
        for u in range(_ILP):
          v = vals_v[b, pl.ds(voff + 16 * u, 16)]
          bk = bidx_v[b, pl.ds(voff + 16 * u, 16)]
          rv = voff + 16 * u + iota
          q1 = jnp.full((16,), -1, jnp.int32)
          for s in range(4):
            half = 8 >> s
            probe = q1 + half
            keys = plsc.load_gather(rows_v, [bb, rv, probe])
            q1 = jnp.where(keys < v, probe, q1)
          pos2 = q1 + 1
          row_at = plsc.load_gather(rows_v, [bb, rv, jnp.minimum(pos2, 15)])
          hit = (pos2 < 16) & (row_at == v)
          miss = lax.rem(v, jnp.full((16,), _ZCH, jnp.int32))
          out_v[b, pl.ds(voff + 16 * u, 16)] = jnp.where(hit, bk * 16 + pos2,
                                                         miss)
        return carry

      lax.fori_loop(0, _CHUNK // (16 * _ILP), grp3, 0)

    def fire_vals(ci, b):
      pltpu.async_copy(values_hbm.at[pl.ds(base + ci * _CHUNK, _CHUNK)],
                       vals_v.at[b], sem_vals)

    def wait_vals(ci, b):
      pltpu.make_async_copy(values_hbm.at[pl.ds(base + ci * _CHUNK, _CHUNK)],
                            vals_v.at[b], sem_vals).wait()

    def fire_out(ci, b):
      pltpu.async_copy(out_v.at[b],
                       out_hbm.at[pl.ds(base + ci * _CHUNK, _CHUNK)], sem_out)

    def drain_out(ci, b):
      pltpu.make_async_copy(out_v.at[b],
                            out_hbm.at[pl.ds(base + ci * _CHUNK, _CHUNK)],
                            sem_out).wait()

    def run():
      pltpu.sync_copy(values_hbm.at[pl.ds(base, _CHUNK)], vals_v.at[0])
      fire_vals(1, 1)
      phase1(0)
      fire_rows(0)

      def body(ci, carry):
        b0 = ci & 1
        b1 = 1 - b0
        wait_vals(ci + 1, b1)
        phase1(b1)
        fire_rows(b1)
        drain_rows(b0)

        @pl.when(ci >= 2)
        def _():
          drain_out(ci, b0)

        phase3(ci, b0)
        fire_out(ci, b0)

        @pl.when(ci + 2 <= nchunks - 1)
        def _():
          fire_vals(ci + 2, b0)

        return carry

      lax.fori_loop(0, nchunks - 1, body, 0)

      bl = (nchunks - 1) & 1
      drain_rows(bl)

      @pl.when(nchunks >= 3)
      def _():
        drain_out(nchunks - 1, bl)

      phase3(nchunks - 1, bl)
      fire_out(nchunks - 1, bl)
      drain_out(nchunks - 2, 1 - bl)
      drain_out(nchunks - 1, bl)
    return run

  for t in range(4):
    pl.when(tid == t)(main_loop(tabs[t]))


_sc_remap = functools.partial(
    pl.kernel,
    out_type=jax.ShapeDtypeStruct((_N,), jnp.int32),
    mesh=plsc.VectorSubcoreMesh(core_axis_name="c", subcore_axis_name="s"),
    scratch_types=[
        pltpu.VMEM((_CPAD,), jnp.int32),
        pltpu.VMEM((_SPAN7, 16), jnp.int32),
        pltpu.VMEM((2, _CHUNK), jnp.int32),
        pltpu.VMEM((2, _CHUNK), jnp.int32),
        pltpu.VMEM((2, _CHUNK, 16), jnp.int32),
        pltpu.VMEM((2, _CHUNK), jnp.int32),
        pltpu.VMEM_SHARED((2, _CPAD), jnp.int32),
        pltpu.SemaphoreType.DMA,
        pltpu.SemaphoreType.DMA,
        pltpu.SemaphoreType.DMA,
    ],
    compiler_params=pltpu.CompilerParams(needs_layout_passes=False,
                                         use_tc_tiling_on_sc=False),
)(_sc_body)


def kernel(values, lengths, weights, mch_sorted_ids_0, mch_sorted_ids_1,
           mch_sorted_ids_2, mch_sorted_ids_3):
  out_values = _sc_remap(values,
                         mch_sorted_ids_0.reshape(_ROWS, 16),
                         mch_sorted_ids_1.reshape(_ROWS, 16),
                         mch_sorted_ids_2.reshape(_ROWS, 16),
                         mch_sorted_ids_3.reshape(_ROWS, 16))
  return out_values, lengths, weights


# speedup vs baseline: 1.5995x; 1.0556x over previous
import functools

import jax
import jax.numpy as jnp
from jax import lax
from jax.experimental import pallas as pl
from jax.experimental.pallas import tpu as pltpu
from jax.experimental.pallas import tpu_sc as plsc

_NUM_FEATURES = 26
_BATCH = 16384
_N = _NUM_FEATURES * _BATCH
_ZCH = 250000
_ROWS = _ZCH // 16
_CPAD = 16384
_IMAX = 2147483647

_NC = 2
_NS = 16

_CHUNK = 1024
_SUB = 128
_NSUB = _CHUNK // _SUB
_ILP = 8

_SPAN = 1952
_SPAN7 = _ROWS - 7 * _SPAN
_CFILL = 15632

_SEG01 = 7 * _BATCH
_SEG23 = 6 * _BATCH
_PER_TILE = _N // 32
_NCH = _PER_TILE // _CHUNK


def _sc_body(values_hbm, m0, m1, m2, m3, out_hbm,
             coarse2, stage_v, vals_v, bidx_v, rows_v, out_v, spmem,
             sem, sem_vals, sem_out):
  tabs = [m0, m1, m2, m3]
  cid = lax.axis_index("c")
  sid = lax.axis_index("s")
  wid = cid * _NS + sid
  q = wid >> 3
  tid = ((q & 1) << 1) | (q >> 1)
  k = wid & 7

  iota = lax.iota(jnp.int32, 16)
  c15 = jnp.full((16,), 15, jnp.int32)
  imax_vec = jnp.full((16,), _IMAX, jnp.int32)

  def extract(m, slot):
    def ex():
      row0 = k * _SPAN
      cref = coarse2.at[slot]

      @pl.when(k < 7)
      def _():
        pltpu.sync_copy(m.at[pl.ds(row0, _SPAN), :],
                        stage_v.at[pl.ds(0, _SPAN), :])

      @pl.when(k == 7)
      def _():
        pltpu.sync_copy(m.at[pl.ds(row0, _SPAN7), :], stage_v)

      nu = jnp.where(k == 7, _SPAN7 // 16 + 1, _SPAN // 16)

      def gx(u, carry):
        keys = plsc.load_gather(stage_v, [u * 16 + iota, c15])
        cref[pl.ds(row0 + u * 16, 16)] = keys
        return carry

      lax.fori_loop(0, nu, gx, 0)

      @pl.when(k == 7)
      def _():
        last = cref[pl.ds(_CFILL - 16, 16)]
        cref[pl.ds(_CFILL - 16, 16)] = jnp.where(
            iota < _ROWS - (_CFILL - 16), last, imax_vec)
        pltpu.sync_copy(cref.at[pl.ds(row0, _CFILL - 7 * _SPAN)],
                        spmem.at[slot, pl.ds(row0, _CFILL - 7 * _SPAN)])

      @pl.when(k < 7)
      def _():
        pltpu.sync_copy(cref.at[pl.ds(row0, _SPAN)],
                        spmem.at[slot, pl.ds(row0, _SPAN)])
    return ex

  for t in range(4):
    pl.when(tid == t)(extract(tabs[t], t >> 1))

  plsc.subcore_barrier()

  for slot in range(2):
    pltpu.sync_copy(spmem.at[slot, pl.ds(0, _CFILL)],
                    coarse2.at[slot, pl.ds(0, _CFILL)])

  def fill(i, carry):
    for slot in range(2):
      coarse2[slot, pl.ds(_CFILL + i * 16, 16)] = imax_vec
    return carry

  lax.fori_loop(0, (_CPAD - _CFILL) // 16, fill, 0)

  ubase = sid * _PER_TILE
  na = jnp.clip((_SEG01 - ubase) // _CHUNK, 0, _NCH)
  nb = _NCH - na
  base_a = ubase + cid * _SEG01
  base_b = ubase + na * _CHUNK + _SEG01 + cid * _SEG23

  def main_loop(m, cref, base, nchunks):
    def phase1(b):
      def grp1(gi, carry):
        voff = gi * (16 * _ILP)
        v = [vals_v[b, pl.ds(voff + 16 * u, 16)] for u in range(_ILP)]
        p1 = [jnp.full((16,), -1, jnp.int32) for _ in range(_ILP)]
        for s in range(14):
          half = _CPAD >> (s + 1)
          probe = [p1[u] + half for u in range(_ILP)]
          keys = [plsc.load_gather(cref, [probe[u]]) for u in range(_ILP)]
          p1 = [jnp.where(keys[u] < v[u], probe[u], p1[u])
                for u in range(_ILP)]
        for u in range(_ILP):
          bidx_v[b, pl.ds(voff + 16 * u, 16)] = jnp.minimum(p1[u] + 1,
                                                            _ROWS - 1)
        return carry

      lax.fori_loop(0, _CHUNK // (16 * _ILP), grp1, 0)

    def fire_rows(b):
      for j in range(_NSUB):
        pltpu.async_copy(m.at[bidx_v.at[b, pl.ds(j * _SUB, _SUB)]],
                         rows_v.at[b, pl.ds(j * _SUB, _SUB)], sem)

    def drain_rows(b):
      for j in range(_NSUB):
        pltpu.make_async_copy(m.at[bidx_v.at[b, pl.ds(j * _SUB, _SUB)]],
                              rows_v.at[b, pl.ds(j * _SUB, _SUB)],
                              sem).wait()

    def phase3(ci, b):
      bb = jnp.full((16,), b, jnp.int32)

      def grp3(gi, carry):
        voff = gi * (16 * _ILP)
        for u in range(_ILP):
          v = vals_v[b, pl.ds(voff + 16 * u, 16)]
          bk = bidx_v[b, pl.ds(voff + 16 * u, 16)]
          rv = voff + 16 * u + iota
          q1 = jnp.full((16,), -1, jnp.int32)
          for s in range(4):
            half = 8 >> s
            probe = q1 + half
            keys = plsc.load_gather(rows_v, [bb, rv, probe])
            q1 = jnp.where(keys < v, probe, q1)
          pos2 = q1 + 1
          row_at = plsc.load_gather(rows_v, [bb, rv, jnp.minimum(pos2, 15)])
          hit = (pos2 < 16) & (row_at == v)
          miss = lax.rem(v, jnp.full((16,), _ZCH, jnp.int32))
          out_v[b, pl.ds(voff + 16 * u, 16)] = jnp.where(hit, bk * 16 + pos2,
                                                         miss)
        return carry

      lax.fori_loop(0, _CHUNK // (16 * _ILP), grp3, 0)

    def fire_vals(ci, b):
      pltpu.async_copy(values_hbm.at[pl.ds(base + ci * _CHUNK, _CHUNK)],
                       vals_v.at[b], sem_vals)

    def wait_vals(ci, b):
      pltpu.make_async_copy(values_hbm.at[pl.ds(base + ci * _CHUNK, _CHUNK)],
                            vals_v.at[b], sem_vals).wait()

    def fire_out(ci, b):
      pltpu.async_copy(out_v.at[b],
                       out_hbm.at[pl.ds(base + ci * _CHUNK, _CHUNK)], sem_out)

    def drain_out(ci, b):
      pltpu.make_async_copy(out_v.at[b],
                            out_hbm.at[pl.ds(base + ci * _CHUNK, _CHUNK)],
                            sem_out).wait()

    def run():
      pltpu.sync_copy(values_hbm.at[pl.ds(base, _CHUNK)], vals_v.at[0])

      @pl.when(nchunks >= 2)
      def _():
        fire_vals(1, 1)

      phase1(0)
      fire_rows(0)

      def body(ci, carry):
        b0 = ci & 1
        b1 = 1 - b0
        wait_vals(ci + 1, b1)
        phase1(b1)
        fire_rows(b1)
        drain_rows(b0)

        @pl.when(ci >= 2)
        def _():
          drain_out(ci, b0)

        phase3(ci, b0)
        fire_out(ci, b0)

        @pl.when(ci + 2 <= nchunks - 1)
        def _():
          fire_vals(ci + 2, b0)

        return carry

      lax.fori_loop(0, nchunks - 1, body, 0)

      bl = (nchunks - 1) & 1
      drain_rows(bl)

      @pl.when(nchunks >= 3)
      def _():
        drain_out(nchunks - 1, bl)

      phase3(nchunks - 1, bl)
      fire_out(nchunks - 1, bl)

      @pl.when(nchunks >= 2)
      def _():
        drain_out(nchunks - 2, 1 - bl)

      drain_out(nchunks - 1, bl)
    return run

  for c in range(2):
    ma, mb = (m0, m2) if c == 0 else (m1, m3)

    @pl.when(cid == c)
    def _(ma=ma, mb=mb):
      pl.when(na > 0)(main_loop(ma, coarse2.at[0], base_a, na))
      pl.when(nb > 0)(main_loop(mb, coarse2.at[1], base_b, nb))


_sc_remap = functools.partial(
    pl.kernel,
    out_type=jax.ShapeDtypeStruct((_N,), jnp.int32),
    mesh=plsc.VectorSubcoreMesh(core_axis_name="c", subcore_axis_name="s"),
    scratch_types=[
        pltpu.VMEM((2, _CPAD), jnp.int32),
        pltpu.VMEM((_SPAN7, 16), jnp.int32),
        pltpu.VMEM((2, _CHUNK), jnp.int32),
        pltpu.VMEM((2, _CHUNK), jnp.int32),
        pltpu.VMEM((2, _CHUNK, 16), jnp.int32),
        pltpu.VMEM((2, _CHUNK), jnp.int32),
        pltpu.VMEM_SHARED((2, _CPAD), jnp.int32),
        pltpu.SemaphoreType.DMA,
        pltpu.SemaphoreType.DMA,
        pltpu.SemaphoreType.DMA,
    ],
    compiler_params=pltpu.CompilerParams(needs_layout_passes=False,
                                         use_tc_tiling_on_sc=False),
)(_sc_body)


def kernel(values, lengths, weights, mch_sorted_ids_0, mch_sorted_ids_1,
           mch_sorted_ids_2, mch_sorted_ids_3):
  out_values = _sc_remap(values,
                         mch_sorted_ids_0.reshape(_ROWS, 16),
                         mch_sorted_ids_1.reshape(_ROWS, 16),
                         mch_sorted_ids_2.reshape(_ROWS, 16),
                         mch_sorted_ids_3.reshape(_ROWS, 16))
  return out_values, lengths, weights
